# baseline (speedup 1.0000x reference)
import jax
import jax.numpy as jnp
from jax.experimental import pallas as pl

N_FD = 50000
E_FD = 800000
L = 24
T = 2048
T_SP = 1024
N_CE = L * T
N_SP = L * T_SP
FD_ITERS = 8


def _fd(fd_xyz, fd_loads, fd_edges, fd_q, pin_idx):
    i, j = fd_edges[0], fd_edges[1]
    diag = jnp.zeros((N_FD,), jnp.float32).at[i].add(fd_q).at[j].add(fd_q) + 1e-6
    x = fd_xyz
    pinned = fd_xyz[pin_idx]
    for _ in range(FD_ITERS):
        s = jnp.zeros((N_FD, 3), jnp.float32).at[i].add(fd_q[:, None] * x[j]).at[j].add(fd_q[:, None] * x[i])
        x = (fd_loads + s) / diag[:, None]
        x = x.at[pin_idx].set(pinned)
    s = jnp.zeros((N_FD, 3), jnp.float32).at[i].add(fd_q[:, None] * x[j]).at[j].add(fd_q[:, None] * x[i])
    residuals = fd_loads + s - diag[:, None] * x
    return x, residuals


def _ce(loads_flat, xyz_flat, lengths, n_trails):
    loads = loads_flat.reshape(L, n_trails, 3)
    origin = xyz_flat.reshape(L, n_trails, 3)[0]
    def step(carry, inp):
        pos, res = carry
        ld, ln = inp
        res = res + ld
        nrm = jnp.sqrt(jnp.sum(res * res, axis=-1, keepdims=True)) + 1e-8
        pos = pos + res / nrm * ln[:, None]
        return (pos, res), pos
    (pos_f, res_f), ys = jax.lax.scan(step, (origin, jnp.zeros((n_trails, 3), jnp.float32)), (loads, lengths))
    return ys.reshape(L * n_trails, 3), res_f


def kernel(fd_xyz, fd_loads, fd_edges, fd_q, indices_fdm, indices_spoke_fdm,
           indices_cem, indices_spoke_cem, cem_loads, cem_xyz, ce_lengths,
           cem2_loads, cem2_xyz, ce_spoke_lengths):
    pin_idx = jnp.concatenate([indices_fdm, indices_spoke_fdm])
    fd_xyz_out, fd_res = _fd(fd_xyz, fd_loads, fd_edges, fd_q, pin_idx)
    fd_reactions = fd_res[indices_fdm, :]
    fd_supports = fd_xyz_out[indices_fdm, :]
    loads1 = cem_loads.at[indices_cem, :].set(fd_reactions)
    xyz1 = cem_xyz.at[indices_cem, :].set(fd_supports)
    ce_xyz, ce_res = _ce(loads1, xyz1, ce_lengths, T)
    fd_spoke_reactions = fd_res[indices_spoke_fdm, :]
    loads2 = cem2_loads.at[indices_spoke_cem, :].set(fd_spoke_reactions)
    spoke_xyz, spoke_res = _ce(loads2, cem2_xyz, ce_spoke_lengths, T_SP)
    return (ce_xyz, ce_res, fd_xyz_out, fd_res, spoke_xyz, spoke_res)


# speedup vs baseline: 21.1008x; 21.1008x over previous
import jax
import jax.numpy as jnp
from jax import lax
from jax.experimental import pallas as pl
from jax.experimental.pallas import tpu as pltpu
from jax.experimental.pallas import tpu_sc as plsc

N_FD = 50000
E_FD = 800000
L = 24
T = 2048
T_SP = 1024
FD_ITERS = 8
N_PIN = 3072

NTILE = 16
COLS = 128
ROWS = 400
N_PAD = ROWS * COLS
ROWS_T = ROWS // NTILE
SGROUPS = ROWS_T * (COLS // 16)
CHUNK = 1280
ECROWS = CHUNK // COLS
NCHUNK = 40
E_TILE = CHUNK * NCHUNK
E_PAD = NTILE * E_TILE
GROUPS = CHUNK // 16
RCHUNKS = 5
RCLEN = ROWS // RCHUNKS
DUMMY = N_PAD - 1


def _fd_body(xs_hbm, loads_hbm, eidx_hbm, eq_hbm, pins_hbm, zeros_hbm,
             idrows_hbm, ownrows_hbm, outx_hbm, outres_hbm,
             x_cur, s_part, ebuf, eqbuf, idrows, ownrows, diag_s, a_s,
             b_s, red_s, spmem_sum):
    c = lax.axis_index("c")
    t = lax.axis_index("s")
    r_off = t * ROWS_T
    ones16 = jnp.full((16,), 1.0, jnp.float32)

    def rc(g):
        return g // 8, (g % 8) * 16

    def zero_s_part_and_sum_slice():
        pltpu.sync_copy(zeros_hbm, s_part)
        pltpu.sync_copy(s_part.at[pl.ds(0, ROWS_T)], spmem_sum.at[ownrows])

    def scatter_pass(with_q_times_x):
        def chunk_body(k, _):
            gchunk = t * NCHUNK + k
            pltpu.sync_copy(eidx_hbm.at[gchunk], ebuf)
            pltpu.sync_copy(eq_hbm.at[gchunk], eqbuf)

            def group_body(g, _):
                r, cb = rc(g)
                sl = pl.ds(cb, 16)
                iv = ebuf[r, sl]
                jv = ebuf[ECROWS + r, sl]
                qv = eqbuf[r, sl]
                ir, ic = iv >> 7, iv & 127
                jr, jc = jv >> 7, jv & 127
                if with_q_times_x:
                    xj = plsc.load_gather(x_cur, [jr, jc])
                    xi = plsc.load_gather(x_cur, [ir, ic])
                    plsc.addupdate_scatter(s_part, [ir, ic], qv * xj)
                    plsc.addupdate_scatter(s_part, [jr, jc], qv * xi)
                else:
                    plsc.addupdate_scatter(s_part, [ir, ic], qv)
                    plsc.addupdate_scatter(s_part, [jr, jc], qv)
                return _

            lax.fori_loop(0, GROUPS, group_body, None)
            return _

        lax.fori_loop(0, NCHUNK, chunk_body, None)

    def reduce_to_slice():
        plsc.subcore_barrier()
        for ch in range(RCHUNKS):
            pltpu.sync_copy(s_part.at[pl.ds(ch * RCLEN, RCLEN)],
                            spmem_sum.at[idrows.at[ch]], add=True)
        plsc.subcore_barrier()
        pltpu.sync_copy(spmem_sum.at[ownrows], red_s)

    pltpu.sync_copy(idrows_hbm, idrows)
    pltpu.sync_copy(ownrows_hbm.at[t], ownrows)

    pltpu.sync_copy(zeros_hbm, s_part)
    pltpu.sync_copy(pins_hbm, ebuf.at[pl.ds(0, N_PIN // COLS)])

    def pin_body(g, _):
        r, cb = rc(g)
        pv = ebuf[r, pl.ds(cb, 16)]
        plsc.addupdate_scatter(s_part, [pv >> 7, pv & 127], ones16)
        return _

    lax.fori_loop(0, N_PIN // 16, pin_body, None)

    def mask_body(g, _):
        r, cb = rc(g)
        a_s[r, pl.ds(cb, 16)] = 1.0 - jnp.minimum(
            s_part[r_off + r, pl.ds(cb, 16)], 1.0)
        return _

    lax.fori_loop(0, SGROUPS, mask_body, None)

    zero_s_part_and_sum_slice()
    plsc.subcore_barrier()
    scatter_pass(with_q_times_x=False)
    reduce_to_slice()

    def diag_body(g, _):
        r, cb = rc(g)
        sl = pl.ds(cb, 16)
        dv = red_s[r, sl] + 1e-6
        diag_s[r, sl] = dv
        a_s[r, sl] = a_s[r, sl] / dv
        return _

    lax.fori_loop(0, SGROUPS, diag_body, None)
    plsc.subcore_barrier()

    for slot in range(2):
        coord = 2 * c + slot

        @pl.when(coord <= 2)
        def _():
            pltpu.sync_copy(xs_hbm.at[coord], x_cur)
            pltpu.sync_copy(loads_hbm.at[coord, t], b_s)

            def b_body(g, _):
                r, cb = rc(g)
                sl = pl.ds(cb, 16)
                m = 1.0 - a_s[r, sl] * diag_s[r, sl]
                x0 = x_cur[r_off + r, sl]
                b_s[r, sl] = m * x0 + b_s[r, sl] * a_s[r, sl]
                return _

            lax.fori_loop(0, SGROUPS, b_body, None)

            for p in range(FD_ITERS + 1):
                zero_s_part_and_sum_slice()
                plsc.subcore_barrier()
                scatter_pass(with_q_times_x=True)
                reduce_to_slice()

                if p < FD_ITERS:
                    def upd_body(g, _):
                        r, cb = rc(g)
                        sl = pl.ds(cb, 16)
                        red_s[r, sl] = b_s[r, sl] + a_s[r, sl] * red_s[r, sl]
                        return _

                    lax.fori_loop(0, SGROUPS, upd_body, None)
                    pltpu.sync_copy(red_s, spmem_sum.at[ownrows])
                    if p == FD_ITERS - 1:
                        pltpu.sync_copy(red_s, outx_hbm.at[coord, t])
                    plsc.subcore_barrier()
                    pltpu.sync_copy(spmem_sum, x_cur)
                else:
                    pltpu.sync_copy(loads_hbm.at[coord, t], b_s)

                    def res_body(g, _):
                        r, cb = rc(g)
                        sl = pl.ds(cb, 16)
                        xv = x_cur[r_off + r, sl]
                        red_s[r, sl] = (b_s[r, sl] + red_s[r, sl]
                                        - diag_s[r, sl] * xv)
                        return _

                    lax.fori_loop(0, SGROUPS, res_body, None)
                    pltpu.sync_copy(red_s, outres_hbm.at[coord, t])


@jax.jit
def _fd_solve(xs, loads, eidx, eq, pins, zeros, idrows, ownrows):
    mesh = plsc.VectorSubcoreMesh(core_axis_name="c", subcore_axis_name="s")
    f = pl.kernel(
        _fd_body,
        out_type=(
            jax.ShapeDtypeStruct((3, NTILE, ROWS_T, COLS), jnp.float32),
            jax.ShapeDtypeStruct((3, NTILE, ROWS_T, COLS), jnp.float32),
        ),
        mesh=mesh,
        compiler_params=pltpu.CompilerParams(needs_layout_passes=False),
        scratch_types=[
            pltpu.VMEM((ROWS, COLS), jnp.float32),
            pltpu.VMEM((ROWS, COLS), jnp.float32),
            pltpu.VMEM((2 * ECROWS, COLS), jnp.int32),
            pltpu.VMEM((ECROWS, COLS), jnp.float32),
            pltpu.VMEM((RCHUNKS, RCLEN), jnp.int32),
            pltpu.VMEM((ROWS_T,), jnp.int32),
            pltpu.VMEM((ROWS_T, COLS), jnp.float32),
            pltpu.VMEM((ROWS_T, COLS), jnp.float32),
            pltpu.VMEM((ROWS_T, COLS), jnp.float32),
            pltpu.VMEM((ROWS_T, COLS), jnp.float32),
            pltpu.VMEM_SHARED((ROWS, COLS), jnp.float32),
        ],
    )
    return f(xs, loads, eidx, eq, pins, zeros, idrows, ownrows)


def _ce(loads_flat, xyz_flat, lengths, n_trails):
    loads = loads_flat.reshape(L, n_trails, 3)
    origin = xyz_flat.reshape(L, n_trails, 3)[0]

    def step(carry, inp):
        pos, res = carry
        ld, ln = inp
        res = res + ld
        nrm = jnp.sqrt(jnp.sum(res * res, axis=-1, keepdims=True)) + 1e-8
        pos = pos + res / nrm * ln[:, None]
        return (pos, res), pos

    (_, res_f), ys = lax.scan(
        step, (origin, jnp.zeros((n_trails, 3), jnp.float32)), (loads, lengths))
    return ys.reshape(L * n_trails, 3), res_f


def kernel(fd_xyz, fd_loads, fd_edges, fd_q, indices_fdm, indices_spoke_fdm,
           indices_cem, indices_spoke_cem, cem_loads, cem_xyz, ce_lengths,
           cem2_loads, cem2_xyz, ce_spoke_lengths):
    xs = jnp.zeros((3, N_PAD), jnp.float32).at[:, :N_FD].set(fd_xyz.T)
    loads = jnp.zeros((3, N_PAD), jnp.float32).at[:, :N_FD].set(fd_loads.T)
    xs = xs.reshape(3, ROWS, COLS)
    loads = loads.reshape(3, NTILE, ROWS_T, COLS)
    ei = jnp.full((E_PAD,), DUMMY, jnp.int32).at[:E_FD].set(fd_edges[0])
    ej = jnp.full((E_PAD,), DUMMY, jnp.int32).at[:E_FD].set(fd_edges[1])
    eqv = jnp.zeros((E_PAD,), jnp.float32).at[:E_FD].set(fd_q)
    ip = ei.reshape(NTILE * NCHUNK, ECROWS, COLS)
    jp = ej.reshape(NTILE * NCHUNK, ECROWS, COLS)
    eidx = jnp.concatenate([ip, jp], axis=1)
    eq = eqv.reshape(NTILE * NCHUNK, ECROWS, COLS)
    pins = jnp.concatenate([indices_fdm, indices_spoke_fdm]).astype(jnp.int32)
    pins = pins.reshape(N_PIN // COLS, COLS)
    zeros = jnp.zeros((ROWS, COLS), jnp.float32)
    idrows = jnp.arange(ROWS, dtype=jnp.int32).reshape(RCHUNKS, RCLEN)
    ownrows = jnp.arange(ROWS, dtype=jnp.int32).reshape(NTILE, ROWS_T)

    outx, outres = _fd_solve(xs, loads, eidx, eq, pins, zeros, idrows, ownrows)
    fd_xyz_out = outx.reshape(3, N_PAD)[:, :N_FD].T
    fd_res = outres.reshape(3, N_PAD)[:, :N_FD].T

    fd_reactions = fd_res[indices_fdm, :]
    fd_supports = fd_xyz_out[indices_fdm, :]
    loads1 = cem_loads.at[indices_cem, :].set(fd_reactions)
    xyz1 = cem_xyz.at[indices_cem, :].set(fd_supports)
    ce_xyz, ce_res = _ce(loads1, xyz1, ce_lengths, T)
    fd_spoke_reactions = fd_res[indices_spoke_fdm, :]
    loads2 = cem2_loads.at[indices_spoke_cem, :].set(fd_spoke_reactions)
    spoke_xyz, spoke_res = _ce(loads2, cem2_xyz, ce_spoke_lengths, T_SP)
    return (ce_xyz, ce_res, fd_xyz_out, fd_res, spoke_xyz, spoke_res)


# speedup vs baseline: 36.6926x; 1.7389x over previous
import jax
import jax.numpy as jnp
from jax import lax
from jax.experimental import pallas as pl
from jax.experimental.pallas import tpu as pltpu
from jax.experimental.pallas import tpu_sc as plsc

N_FD = 50000
E_FD = 800000
L = 24
T = 2048
T_SP = 1024
FD_ITERS = 8
N_PIN = 3072
N_PIN_PAD = 4096

NTILE = 16
COLS = 128
ROWS = 400
N_PAD = ROWS * COLS
ROWS_T = ROWS // NTILE
SGROUPS = ROWS_T * (COLS // 16)
CHUNK = 1280
ECROWS = CHUNK // COLS
NCHUNK = 40
E_TILE = CHUNK * NCHUNK
E_PAD = NTILE * E_TILE
GROUPS = CHUNK // 16
RCHUNKS = 5
RCLEN = ROWS // RCHUNKS
DUMMY = N_PAD - 1


def _fd_body(xs_hbm, loads_hbm, epack_hbm, pins_hbm, zeros_hbm,
             idrows_hbm, ownrows_hbm, outx_hbm, outres_hbm,
             x_cur, s_part, ebuf0, ebuf1, idrows, ownrows, diag_s, a_s,
             b_s, red_s, esem0, esem1, zsem, spmem_sum):
    c = lax.axis_index("c")
    t = lax.axis_index("s")
    r_off = t * ROWS_T
    ones16 = jnp.full((16,), 1.0, jnp.float32)

    def rc(g):
        return g // 8, (g % 8) * 16

    def start_chunk(k, eb, sem):
        pltpu.async_copy(epack_hbm.at[t * NCHUNK + k], eb, sem)

    def wait_chunk(k, eb, sem):
        pltpu.make_async_copy(epack_hbm.at[t * NCHUNK + k], eb, sem).wait()

    def compute_chunk(eb, with_q_times_x):
        @plsc.parallel_loop(0, GROUPS, unroll=4)
        def _(g):
            r, cb = rc(g)
            sl = pl.ds(cb, 16)
            v = eb[r, sl]
            qv = plsc.bitcast(eb[ECROWS + r, sl], jnp.float32)
            iv = v & 0xFFFF
            jv = lax.shift_right_logical(v, 16)
            ir, ic = iv >> 7, iv & 127
            jr, jc = jv >> 7, jv & 127
            if with_q_times_x:
                xj = plsc.load_gather(x_cur, [jr, jc])
                xi = plsc.load_gather(x_cur, [ir, ic])
                plsc.addupdate_scatter(s_part, [ir, ic], qv * xj)
                plsc.addupdate_scatter(s_part, [jr, jc], qv * xi)
            else:
                plsc.addupdate_scatter(s_part, [ir, ic], qv)
                plsc.addupdate_scatter(s_part, [jr, jc], qv)

    def scatter_pass(with_q_times_x):
        start_chunk(0, ebuf0, esem0)
        zd = pltpu.async_copy(zeros_hbm, s_part, zsem)
        zd.wait()
        pltpu.sync_copy(s_part.at[pl.ds(0, ROWS_T)], spmem_sum.at[ownrows])

        def pair_body(kk, _):
            k0 = 2 * kk
            start_chunk(k0 + 1, ebuf1, esem1)
            wait_chunk(k0, ebuf0, esem0)
            compute_chunk(ebuf0, with_q_times_x)

            @pl.when(k0 + 2 < NCHUNK)
            def _():
                start_chunk(k0 + 2, ebuf0, esem0)

            wait_chunk(k0 + 1, ebuf1, esem1)
            compute_chunk(ebuf1, with_q_times_x)
            return _

        lax.fori_loop(0, NCHUNK // 2, pair_body, None)

    def reduce_to_slice():
        plsc.subcore_barrier()
        for ch in range(RCHUNKS):
            pltpu.sync_copy(s_part.at[pl.ds(ch * RCLEN, RCLEN)],
                            spmem_sum.at[idrows.at[ch]], add=True)
        plsc.subcore_barrier()
        pltpu.sync_copy(spmem_sum.at[ownrows], red_s)

    pltpu.sync_copy(idrows_hbm, idrows)
    pltpu.sync_copy(ownrows_hbm.at[t], ownrows)

    pltpu.sync_copy(zeros_hbm, s_part)
    pltpu.sync_copy(pins_hbm.at[0], ebuf0.at[pl.ds(0, 16)])
    pltpu.sync_copy(pins_hbm.at[1], ebuf1.at[pl.ds(0, 16)])

    for eb in (ebuf0, ebuf1):
        def pin_body(g, _, eb=eb):
            r, cb = rc(g)
            pv = eb[r, pl.ds(cb, 16)]
            plsc.addupdate_scatter(s_part, [pv >> 7, pv & 127], ones16)
            return _

        lax.fori_loop(0, (N_PIN_PAD // 2) // 16, pin_body, None)

    def mask_body(g, _):
        r, cb = rc(g)
        a_s[r, pl.ds(cb, 16)] = 1.0 - jnp.minimum(
            s_part[r_off + r, pl.ds(cb, 16)], 1.0)
        return _

    lax.fori_loop(0, SGROUPS, mask_body, None)

    scatter_pass(with_q_times_x=False)
    reduce_to_slice()

    def diag_body(g, _):
        r, cb = rc(g)
        sl = pl.ds(cb, 16)
        dv = red_s[r, sl] + 1e-6
        diag_s[r, sl] = dv
        a_s[r, sl] = a_s[r, sl] / dv
        return _

    lax.fori_loop(0, SGROUPS, diag_body, None)
    plsc.subcore_barrier()

    for slot in range(2):
        coord = 2 * c + slot

        @pl.when(coord <= 2)
        def _():
            pltpu.sync_copy(xs_hbm.at[coord], x_cur)
            pltpu.sync_copy(loads_hbm.at[coord, t], b_s)

            def b_body(g, _):
                r, cb = rc(g)
                sl = pl.ds(cb, 16)
                m = 1.0 - a_s[r, sl] * diag_s[r, sl]
                x0 = x_cur[r_off + r, sl]
                b_s[r, sl] = m * x0 + b_s[r, sl] * a_s[r, sl]
                return _

            lax.fori_loop(0, SGROUPS, b_body, None)
            plsc.subcore_barrier()

            for p in range(FD_ITERS + 1):
                scatter_pass(with_q_times_x=True)
                reduce_to_slice()

                if p < FD_ITERS:
                    def upd_body(g, _):
                        r, cb = rc(g)
                        sl = pl.ds(cb, 16)
                        red_s[r, sl] = b_s[r, sl] + a_s[r, sl] * red_s[r, sl]
                        return _

                    lax.fori_loop(0, SGROUPS, upd_body, None)
                    pltpu.sync_copy(red_s, spmem_sum.at[ownrows])
                    if p == FD_ITERS - 1:
                        pltpu.sync_copy(red_s, outx_hbm.at[coord, t])
                    plsc.subcore_barrier()
                    pltpu.sync_copy(spmem_sum, x_cur)
                    plsc.subcore_barrier()
                else:
                    pltpu.sync_copy(loads_hbm.at[coord, t], b_s)

                    def res_body(g, _):
                        r, cb = rc(g)
                        sl = pl.ds(cb, 16)
                        xv = x_cur[r_off + r, sl]
                        red_s[r, sl] = (b_s[r, sl] + red_s[r, sl]
                                        - diag_s[r, sl] * xv)
                        return _

                    lax.fori_loop(0, SGROUPS, res_body, None)
                    pltpu.sync_copy(red_s, outres_hbm.at[coord, t])


@jax.jit
def _fd_solve(xs, loads, epack, pins, zeros, idrows, ownrows):
    mesh = plsc.VectorSubcoreMesh(core_axis_name="c", subcore_axis_name="s")
    f = pl.kernel(
        _fd_body,
        out_type=(
            jax.ShapeDtypeStruct((3, NTILE, ROWS_T, COLS), jnp.float32),
            jax.ShapeDtypeStruct((3, NTILE, ROWS_T, COLS), jnp.float32),
        ),
        mesh=mesh,
        compiler_params=pltpu.CompilerParams(needs_layout_passes=False),
        scratch_types=[
            pltpu.VMEM((ROWS, COLS), jnp.float32),
            pltpu.VMEM((ROWS, COLS), jnp.float32),
            pltpu.VMEM((2 * ECROWS, COLS), jnp.int32),
            pltpu.VMEM((2 * ECROWS, COLS), jnp.int32),
            pltpu.VMEM((RCHUNKS, RCLEN), jnp.int32),
            pltpu.VMEM((ROWS_T,), jnp.int32),
            pltpu.VMEM((ROWS_T, COLS), jnp.float32),
            pltpu.VMEM((ROWS_T, COLS), jnp.float32),
            pltpu.VMEM((ROWS_T, COLS), jnp.float32),
            pltpu.VMEM((ROWS_T, COLS), jnp.float32),
            pltpu.SemaphoreType.DMA,
            pltpu.SemaphoreType.DMA,
            pltpu.SemaphoreType.DMA,
            pltpu.VMEM_SHARED((ROWS, COLS), jnp.float32),
        ],
    )
    return f(xs, loads, epack, pins, zeros, idrows, ownrows)


def _ce(loads_flat, xyz_flat, lengths, n_trails):
    loads = loads_flat.reshape(L, n_trails, 3)
    origin = xyz_flat.reshape(L, n_trails, 3)[0]

    def step(carry, inp):
        pos, res = carry
        ld, ln = inp
        res = res + ld
        nrm = jnp.sqrt(jnp.sum(res * res, axis=-1, keepdims=True)) + 1e-8
        pos = pos + res / nrm * ln[:, None]
        return (pos, res), pos

    (_, res_f), ys = lax.scan(
        step, (origin, jnp.zeros((n_trails, 3), jnp.float32)), (loads, lengths))
    return ys.reshape(L * n_trails, 3), res_f


def kernel(fd_xyz, fd_loads, fd_edges, fd_q, indices_fdm, indices_spoke_fdm,
           indices_cem, indices_spoke_cem, cem_loads, cem_xyz, ce_lengths,
           cem2_loads, cem2_xyz, ce_spoke_lengths):
    xs = jnp.zeros((3, N_PAD), jnp.float32).at[:, :N_FD].set(fd_xyz.T)
    loads = jnp.zeros((3, N_PAD), jnp.float32).at[:, :N_FD].set(fd_loads.T)
    xs = xs.reshape(3, ROWS, COLS)
    loads = loads.reshape(3, NTILE, ROWS_T, COLS)
    ei = jnp.full((E_PAD,), DUMMY, jnp.int32).at[:E_FD].set(fd_edges[0])
    ej = jnp.full((E_PAD,), DUMMY, jnp.int32).at[:E_FD].set(fd_edges[1])
    eqv = jnp.zeros((E_PAD,), jnp.float32).at[:E_FD].set(fd_q)
    packed = (ej << 16) | ei
    pk = packed.reshape(NTILE * NCHUNK, ECROWS, COLS)
    qk = lax.bitcast_convert_type(eqv, jnp.int32).reshape(
        NTILE * NCHUNK, ECROWS, COLS)
    epack = jnp.concatenate([pk, qk], axis=1)
    pins = jnp.concatenate([indices_fdm, indices_spoke_fdm]).astype(jnp.int32)
    pins = jnp.full((N_PIN_PAD,), DUMMY, jnp.int32).at[:N_PIN].set(pins)
    pins = pins.reshape(2, 16, COLS)
    zeros = jnp.zeros((ROWS, COLS), jnp.float32)
    idrows = jnp.arange(ROWS, dtype=jnp.int32).reshape(RCHUNKS, RCLEN)
    ownrows = jnp.arange(ROWS, dtype=jnp.int32).reshape(NTILE, ROWS_T)

    outx, outres = _fd_solve(xs, loads, epack, pins, zeros, idrows, ownrows)
    fd_xyz_out = outx.reshape(3, N_PAD)[:, :N_FD].T
    fd_res = outres.reshape(3, N_PAD)[:, :N_FD].T

    fd_reactions = fd_res[indices_fdm, :]
    fd_supports = fd_xyz_out[indices_fdm, :]
    loads1 = cem_loads.at[indices_cem, :].set(fd_reactions)
    xyz1 = cem_xyz.at[indices_cem, :].set(fd_supports)
    ce_xyz, ce_res = _ce(loads1, xyz1, ce_lengths, T)
    fd_spoke_reactions = fd_res[indices_spoke_fdm, :]
    loads2 = cem2_loads.at[indices_spoke_cem, :].set(fd_spoke_reactions)
    spoke_xyz, spoke_res = _ce(loads2, cem2_xyz, ce_spoke_lengths, T_SP)
    return (ce_xyz, ce_res, fd_xyz_out, fd_res, spoke_xyz, spoke_res)
